# baseline (speedup 1.0000x reference)
import functools

import jax
import jax.numpy as jnp
from jax import lax
from jax.experimental import pallas as pl
from jax.experimental.pallas import tpu as pltpu
from jax.experimental.pallas import tpu_sc as plsc

_NC = 2
_NS = 16
_NW = _NC * _NS
_D = 64
_V = 1000000
_MFULL = _V // 128
_PAIRS = _V // 2
_L = 16

_mesh = plsc.VectorSubcoreMesh(core_axis_name="c", subcore_axis_name="s")
_params = pltpu.CompilerParams(
    use_tc_tiling_on_sc=True, needs_layout_passes=False
)


def _iota16():
    return lax.iota(jnp.int32, _L)


@functools.partial(
    pl.kernel,
    mesh=_mesh,
    out_type=jax.ShapeDtypeStruct((_PAIRS, 128), jnp.float32),
    scratch_types=[
        pltpu.VMEM((2, 64, 128), jnp.float32),
        pltpu.VMEM((2, 64, 128), jnp.float32),
        pltpu.VMEM((32, 128), jnp.float32),
        pltpu.SemaphoreType.DMA,
        pltpu.SemaphoreType.DMA,
        pltpu.SemaphoreType.DMA,
    ],
    compiler_params=_params,
)
def _transpose_table(vT_hbm, tail2_hbm, vlin2_hbm, buf, tbuf, tailb, rsem, wsem0, wsem1):
    wid = lax.axis_index("s") * _NC + lax.axis_index("c")
    wsems = (wsem0, wsem1)
    it16 = _iota16()
    n_slots = 246

    pltpu.async_copy(vT_hbm.at[:, pl.ds(wid * 128, 128)], buf.at[0], rsem)

    def outer(i, carry):
        for b in range(2):
            slot = 2 * i + b
            m = wid + _NW * slot

            @pl.when(m < _MFULL)
            def _():
                pltpu.make_async_copy(
                    vT_hbm.at[:, pl.ds(0, 128)], buf.at[b], rsem
                ).wait()

                @pl.when(m + _NW < _MFULL)
                def _():
                    pltpu.async_copy(
                        vT_hbm.at[:, pl.ds((m + _NW) * 128, 128)],
                        buf.at[1 - b],
                        rsem,
                    )

                @pl.when(slot >= 2)
                def _():
                    pltpu.make_async_copy(
                        tbuf.at[b], vlin2_hbm.at[pl.ds(0, 64)], wsems[b]
                    ).wait()

                for w0b in range(8):
                    w16 = it16 + 16 * w0b
                    p16 = lax.shift_right_logical(w16, 1)
                    c64 = lax.shift_left(lax.bitwise_and(w16, 1), 6)

                    @plsc.parallel_loop(0, 64, unroll=16)
                    def diag(k):
                        d16 = lax.bitwise_and(it16 + k, 15) + lax.bitwise_and(
                            k, 48
                        )
                        vals = plsc.load_gather(buf.at[b], [d16, w16])
                        plsc.store_scatter(
                            tbuf.at[b], [p16, c64 + d16], vals
                        )

                pltpu.async_copy(
                    tbuf.at[b], vlin2_hbm.at[pl.ds(64 * m, 64)], wsems[b]
                )
        return carry

    lax.fori_loop(0, n_slots // 2, outer, 0)

    for b in range(2):
        pltpu.make_async_copy(
            tbuf.at[b], vlin2_hbm.at[pl.ds(0, 64)], wsems[b]
        ).wait()

    @pl.when(wid == 0)
    def _():
        pltpu.sync_copy(tail2_hbm, tailb)
        pltpu.sync_copy(tailb, vlin2_hbm.at[pl.ds(_PAIRS - 32, 32)])


@functools.partial(
    pl.kernel,
    mesh=_mesh,
    out_type=jax.ShapeDtypeStruct((200, 8, 32, 8, 128), jnp.float32),
    scratch_types=[
        pltpu.VMEM((8, 128), jnp.int32),
        pltpu.VMEM((8, 128), jnp.int32),
        pltpu.VMEM((2, 256, 128), jnp.float32),
        pltpu.VMEM((128, 128), jnp.float32),
        pltpu.SemaphoreType.DMA,
        pltpu.SemaphoreType.DMA,
        pltpu.SemaphoreType.DMA,
    ],
    compiler_params=_params,
)
def _pair_gather(sT_hbm, vlin2_hbm, out_hbm, sidx, pidx, src, dst, gsem, wsem0, wsem1):
    wid = lax.axis_index("s") * _NC + lax.axis_index("c")
    wsems = (wsem0, wsem1)
    it16 = _iota16()

    def tile_body(j, carry):
        n = wid * 25 + j
        tt = lax.shift_right_logical(n, 5)
        bb = lax.bitwise_and(n, 31)

        pltpu.sync_copy(
            sT_hbm.at[pl.ds(tt * 8, 8), pl.ds(bb * 128, 128)], sidx
        )

        @plsc.parallel_loop(0, 8, unroll=2)
        def mkpair(r):
            for k8 in range(8):
                v = sidx[r, pl.ds(16 * k8, 16)]
                pidx[r, pl.ds(16 * k8, 16)] = lax.shift_right_logical(v, 1)

        def fire(sb, q):
            return [
                pltpu.async_copy(
                    vlin2_hbm.at[pidx.at[2 * sb + t]],
                    src.at[q, pl.ds(128 * t, 128)],
                    gsem,
                )
                for t in range(2)
            ]

        pending = fire(0, 0)
        for sb in range(4):
            q = sb & 1
            for cp in pending:
                cp.wait()
            if sb < 3:
                pending = fire(sb + 1, 1 - q)

            @pl.when(jnp.logical_or(j > 0, sb >= 1))
            def _():
                for w in range(16):
                    pltpu.make_async_copy(
                        dst.at[pl.ds(8 * w, 8)], out_hbm.at[0, 0, 0], wsem0
                    ).wait()

            for t in range(2):
                for bg in range(8):
                    rowv = it16 + (128 * t + 16 * bg)
                    colb = it16 + 16 * bg
                    vodd = lax.bitwise_and(
                        sidx[2 * sb + t, pl.ds(16 * bg, 16)], 1
                    )
                    cbase = vodd * _D

                    @plsc.parallel_loop(0, 64, unroll=16)
                    def diag(k):
                        d16 = lax.bitwise_and(it16 + k, 63)
                        vals = plsc.load_gather(
                            src.at[q], [rowv, cbase + d16]
                        )
                        plsc.store_scatter(
                            dst, [t * 64 + d16, colb], vals
                        )

            for t in range(2):
                for dd in range(8):
                    pltpu.async_copy(
                        dst.at[pl.ds(t * 64 + 8 * dd, 8)],
                        out_hbm.at[tt * 8 + 2 * sb + t, dd, bb],
                        wsem0,
                    )
        return carry

    lax.fori_loop(0, 25, tile_body, 0)

    for w in range(16):
        pltpu.make_async_copy(
            dst.at[pl.ds(8 * w, 8)], out_hbm.at[0, 0, 0], wsem0
        ).wait()


def kernel(s, vocab):
    vT = vocab.T
    sT = s.T
    tail2 = vocab[_MFULL * 128 :].reshape(32, 128)
    vlin2 = _transpose_table(vT, tail2)
    out5d = _pair_gather(sT.astype(jnp.int32), vlin2)
    return out5d.transpose(2, 4, 0, 1, 3).reshape(4096, 200, _D)


# speedup vs baseline: 1.2816x; 1.2816x over previous
import functools

import jax
import jax.numpy as jnp
from jax import lax
from jax.experimental import pallas as pl
from jax.experimental.pallas import tpu as pltpu
from jax.experimental.pallas import tpu_sc as plsc

_NC = 2
_NS = 16
_NW = _NC * _NS
_D = 64
_V = 1000000
_MFULL = _V // 128
_PAIRS = _V // 2
_L = 16

_mesh = plsc.VectorSubcoreMesh(core_axis_name="c", subcore_axis_name="s")
_params = pltpu.CompilerParams(
    use_tc_tiling_on_sc=True, needs_layout_passes=False
)


def _iota16():
    return lax.iota(jnp.int32, _L)


@functools.partial(
    pl.kernel,
    mesh=_mesh,
    out_type=jax.ShapeDtypeStruct((_PAIRS, 128), jnp.float32),
    scratch_types=[
        pltpu.VMEM((2, 64, 128), jnp.float32),
        pltpu.VMEM((2, 64, 128), jnp.float32),
        pltpu.VMEM((32, 128), jnp.float32),
        pltpu.SemaphoreType.DMA,
        pltpu.SemaphoreType.DMA,
        pltpu.SemaphoreType.DMA,
    ],
    compiler_params=_params,
)
def _transpose_table(vT_hbm, tail2_hbm, vlin2_hbm, buf, tbuf, tailb, rsem, wsem0, wsem1):
    wid = lax.axis_index("s") * _NC + lax.axis_index("c")
    wsems = (wsem0, wsem1)
    it16 = _iota16()
    n_slots = 246

    pltpu.async_copy(vT_hbm.at[:, pl.ds(wid * 128, 128)], buf.at[0], rsem)

    def outer(i, carry):
        for b in range(2):
            slot = 2 * i + b
            m = wid + _NW * slot

            @pl.when(m < _MFULL)
            def _():
                pltpu.make_async_copy(
                    vT_hbm.at[:, pl.ds(0, 128)], buf.at[b], rsem
                ).wait()

                @pl.when(m + _NW < _MFULL)
                def _():
                    pltpu.async_copy(
                        vT_hbm.at[:, pl.ds((m + _NW) * 128, 128)],
                        buf.at[1 - b],
                        rsem,
                    )

                @pl.when(slot >= 2)
                def _():
                    pltpu.make_async_copy(
                        tbuf.at[b], vlin2_hbm.at[pl.ds(0, 64)], wsems[b]
                    ).wait()

                for w0b in range(8):
                    w16 = it16 + 16 * w0b
                    p16 = lax.shift_right_logical(w16, 1)
                    c64 = lax.shift_left(lax.bitwise_and(w16, 1), 6)

                    @plsc.parallel_loop(0, 64, unroll=8)
                    def diag(k):
                        d16 = lax.bitwise_and(it16 + k, 15) + lax.bitwise_and(
                            k, 48
                        )
                        vals = plsc.load_gather(buf.at[b], [d16, w16])
                        plsc.store_scatter(
                            tbuf.at[b], [p16, c64 + d16], vals
                        )

                pltpu.async_copy(
                    tbuf.at[b], vlin2_hbm.at[pl.ds(64 * m, 64)], wsems[b]
                )
        return carry

    lax.fori_loop(0, n_slots // 2, outer, 0)

    for b in range(2):
        pltpu.make_async_copy(
            tbuf.at[b], vlin2_hbm.at[pl.ds(0, 64)], wsems[b]
        ).wait()

    @pl.when(wid == 0)
    def _():
        pltpu.sync_copy(tail2_hbm, tailb)
        pltpu.sync_copy(tailb, vlin2_hbm.at[pl.ds(_PAIRS - 32, 32)])


@functools.partial(
    pl.kernel,
    mesh=_mesh,
    out_type=jax.ShapeDtypeStruct((200, 8, 32, 8, 128), jnp.float32),
    scratch_types=[
        pltpu.VMEM((8, 128), jnp.int32),
        pltpu.VMEM((8, 128), jnp.int32),
        pltpu.VMEM((2, 256, 128), jnp.float32),
        pltpu.VMEM((128, 128), jnp.float32),
        pltpu.SemaphoreType.DMA,
        pltpu.SemaphoreType.DMA,
        pltpu.SemaphoreType.DMA,
    ],
    compiler_params=_params,
)
def _pair_gather(sT_hbm, vlin2_hbm, out_hbm, sidx, pidx, src, dst, gsem, wsem0, wsem1):
    wid = lax.axis_index("s") * _NC + lax.axis_index("c")
    wsems = (wsem0, wsem1)
    it16 = _iota16()

    def tile_body(j, carry):
        n = wid * 25 + j
        tt = lax.shift_right_logical(n, 5)
        bb = lax.bitwise_and(n, 31)

        pltpu.sync_copy(
            sT_hbm.at[pl.ds(tt * 8, 8), pl.ds(bb * 128, 128)], sidx
        )

        @plsc.parallel_loop(0, 8, unroll=2)
        def mkpair(r):
            for k8 in range(8):
                v = sidx[r, pl.ds(16 * k8, 16)]
                pidx[r, pl.ds(16 * k8, 16)] = lax.shift_right_logical(v, 1)

        def fire(sb, q):
            return [
                pltpu.async_copy(
                    vlin2_hbm.at[pidx.at[2 * sb + t]],
                    src.at[q, pl.ds(128 * t, 128)],
                    gsem,
                )
                for t in range(2)
            ]

        pending = fire(0, 0)
        for sb in range(4):
            q = sb & 1
            for cp in pending:
                cp.wait()
            if sb < 3:
                pending = fire(sb + 1, 1 - q)

            @pl.when(jnp.logical_or(j > 0, sb >= 1))
            def _():
                for w in range(16):
                    pltpu.make_async_copy(
                        dst.at[pl.ds(8 * w, 8)], out_hbm.at[0, 0, 0], wsem0
                    ).wait()

            for t in range(2):
                for bg in range(8):
                    rowv = it16 + (128 * t + 16 * bg)
                    colb = it16 + 16 * bg
                    vodd = lax.bitwise_and(
                        sidx[2 * sb + t, pl.ds(16 * bg, 16)], 1
                    )
                    cbase = vodd * _D

                    @plsc.parallel_loop(0, 64, unroll=8)
                    def diag(k):
                        d16 = lax.bitwise_and(it16 + k, 63)
                        vals = plsc.load_gather(
                            src.at[q], [rowv, cbase + d16]
                        )
                        plsc.store_scatter(
                            dst, [t * 64 + d16, colb], vals
                        )

            for t in range(2):
                for dd in range(8):
                    pltpu.async_copy(
                        dst.at[pl.ds(t * 64 + 8 * dd, 8)],
                        out_hbm.at[tt * 8 + 2 * sb + t, dd, bb],
                        wsem0,
                    )
        return carry

    lax.fori_loop(0, 25, tile_body, 0)

    for w in range(16):
        pltpu.make_async_copy(
            dst.at[pl.ds(8 * w, 8)], out_hbm.at[0, 0, 0], wsem0
        ).wait()


def kernel(s, vocab):
    vT = vocab.T
    sT = s.T
    tail2 = vocab[_MFULL * 128 :].reshape(32, 128)
    vlin2 = _transpose_table(vT, tail2)
    out5d = _pair_gather(sT.astype(jnp.int32), vlin2)
    return out5d.transpose(2, 4, 0, 1, 3).reshape(4096, 200, _D)


# speedup vs baseline: 1.3408x; 1.0463x over previous
import functools

import jax
import jax.numpy as jnp
from jax import lax
from jax.experimental import pallas as pl
from jax.experimental.pallas import tpu as pltpu
from jax.experimental.pallas import tpu_sc as plsc

_NC = 2
_NS = 16
_NW = _NC * _NS
_D = 64
_V = 1000000
_MFULL = _V // 128
_PAIRS = _V // 2
_L = 16

_mesh = plsc.VectorSubcoreMesh(core_axis_name="c", subcore_axis_name="s")
_params = pltpu.CompilerParams(
    use_tc_tiling_on_sc=True, needs_layout_passes=False
)


def _iota16():
    return lax.iota(jnp.int32, _L)


@functools.partial(
    pl.kernel,
    mesh=_mesh,
    out_type=jax.ShapeDtypeStruct((_PAIRS, 128), jnp.float32),
    scratch_types=[
        pltpu.VMEM((2, 64, 128), jnp.float32),
        pltpu.VMEM((2, 64, 128), jnp.float32),
        pltpu.VMEM((32, 128), jnp.float32),
        pltpu.SemaphoreType.DMA,
        pltpu.SemaphoreType.DMA,
        pltpu.SemaphoreType.DMA,
    ],
    compiler_params=_params,
)
def _transpose_table(vT_hbm, tail2_hbm, vlin2_hbm, buf, tbuf, tailb, rsem, wsem0, wsem1):
    wid = lax.axis_index("s") * _NC + lax.axis_index("c")
    wsems = (wsem0, wsem1)
    it16 = _iota16()
    n_slots = 246

    pltpu.async_copy(vT_hbm.at[:, pl.ds(wid * 128, 128)], buf.at[0], rsem)

    def outer(i, carry):
        for b in range(2):
            slot = 2 * i + b
            m = wid + _NW * slot

            @pl.when(m < _MFULL)
            def _():
                pltpu.make_async_copy(
                    vT_hbm.at[:, pl.ds(0, 128)], buf.at[b], rsem
                ).wait()

                @pl.when(m + _NW < _MFULL)
                def _():
                    pltpu.async_copy(
                        vT_hbm.at[:, pl.ds((m + _NW) * 128, 128)],
                        buf.at[1 - b],
                        rsem,
                    )

                @pl.when(slot >= 2)
                def _():
                    pltpu.make_async_copy(
                        tbuf.at[b], vlin2_hbm.at[pl.ds(0, 64)], wsems[b]
                    ).wait()

                for w0b in range(8):
                    w16 = it16 + 16 * w0b
                    p16 = lax.shift_right_logical(w16, 1)
                    c64 = lax.shift_left(lax.bitwise_and(w16, 1), 6)

                    @plsc.parallel_loop(0, 64, unroll=8)
                    def diag(k):
                        d16 = lax.bitwise_and(it16 + k, 15) + lax.bitwise_and(
                            k, 48
                        )
                        vals = plsc.load_gather(buf.at[b], [d16, w16])
                        plsc.store_scatter(
                            tbuf.at[b], [p16, c64 + d16], vals
                        )

                pltpu.async_copy(
                    tbuf.at[b], vlin2_hbm.at[pl.ds(64 * m, 64)], wsems[b]
                )
        return carry

    lax.fori_loop(0, n_slots // 2, outer, 0)

    for b in range(2):
        pltpu.make_async_copy(
            tbuf.at[b], vlin2_hbm.at[pl.ds(0, 64)], wsems[b]
        ).wait()

    @pl.when(wid == 0)
    def _():
        pltpu.sync_copy(tail2_hbm, tailb)
        pltpu.sync_copy(tailb, vlin2_hbm.at[pl.ds(_PAIRS - 32, 32)])


@functools.partial(
    pl.kernel,
    mesh=_mesh,
    out_type=jax.ShapeDtypeStruct((200, 8, 32, 8, 128), jnp.float32),
    scratch_types=[
        pltpu.VMEM((2, 8, 128), jnp.int32),
        pltpu.VMEM((2, 8, 128), jnp.int32),
        pltpu.VMEM((2, 256, 128), jnp.float32),
        pltpu.VMEM((128, 128), jnp.float32),
        pltpu.SemaphoreType.DMA,
        pltpu.SemaphoreType.DMA,
        pltpu.SemaphoreType.DMA,
    ],
    compiler_params=_params,
)
def _pair_gather(sT_hbm, vlin2_hbm, out_hbm, sidx, pidx, src, dst, gsem, wsem0, wsem1):
    wid = lax.axis_index("s") * _NC + lax.axis_index("c")
    wsems = (wsem0, wsem1)
    it16 = _iota16()

    def stage_idx(j, e):
        n = wid * 25 + j
        tt = lax.shift_right_logical(n, 5)
        bb = lax.bitwise_and(n, 31)
        pltpu.sync_copy(
            sT_hbm.at[pl.ds(tt * 8, 8), pl.ds(bb * 128, 128)], sidx.at[e]
        )

        @plsc.parallel_loop(0, 8, unroll=2)
        def mkpair(r):
            for k8 in range(8):
                v = sidx[e, r, pl.ds(16 * k8, 16)]
                pidx[e, r, pl.ds(16 * k8, 16)] = lax.shift_right_logical(v, 1)

    def fire(e, sb, q):
        for t in range(2):
            pltpu.async_copy(
                vlin2_hbm.at[pidx.at[e, 2 * sb + t]],
                src.at[q, pl.ds(128 * t, 128)],
                gsem,
            )

    def drain_gathers(q):
        for t in range(2):
            pltpu.make_async_copy(
                vlin2_hbm.at[pidx.at[0, 0]],
                src.at[q, pl.ds(128 * t, 128)],
                gsem,
            ).wait()

    stage_idx(0, 0)
    fire(0, 0, 0)

    def tile_body(j, carry):
        e = lax.bitwise_and(j, 1)
        n = wid * 25 + j
        tt = lax.shift_right_logical(n, 5)
        bb = lax.bitwise_and(n, 31)

        for sb in range(4):
            q = sb & 1
            drain_gathers(q)
            if sb < 3:
                fire(e, sb + 1, 1 - q)

            @pl.when(jnp.logical_or(j > 0, sb >= 1))
            def _():
                for w in range(16):
                    pltpu.make_async_copy(
                        dst.at[pl.ds(8 * w, 8)], out_hbm.at[0, 0, 0], wsem0
                    ).wait()

            if sb == 3:

                @pl.when(j < 24)
                def _():
                    stage_idx(j + 1, 1 - e)
                    fire(1 - e, 0, 0)

            for t in range(2):
                for bg in range(8):
                    rowv = it16 + (128 * t + 16 * bg)
                    colb = it16 + 16 * bg
                    vodd = lax.bitwise_and(
                        sidx[e, 2 * sb + t, pl.ds(16 * bg, 16)], 1
                    )
                    cbase = vodd * _D

                    @plsc.parallel_loop(0, 64, unroll=8)
                    def diag(k):
                        d16 = lax.bitwise_and(it16 + k, 63)
                        vals = plsc.load_gather(
                            src.at[q], [rowv, cbase + d16]
                        )
                        plsc.store_scatter(
                            dst, [t * 64 + d16, colb], vals
                        )

            for t in range(2):
                for dd in range(8):
                    pltpu.async_copy(
                        dst.at[pl.ds(t * 64 + 8 * dd, 8)],
                        out_hbm.at[tt * 8 + 2 * sb + t, dd, bb],
                        wsem0,
                    )
        return carry

    lax.fori_loop(0, 25, tile_body, 0)

    for w in range(16):
        pltpu.make_async_copy(
            dst.at[pl.ds(8 * w, 8)], out_hbm.at[0, 0, 0], wsem0
        ).wait()


def kernel(s, vocab):
    vT = vocab.T
    sT = s.T
    tail2 = vocab[_MFULL * 128 :].reshape(32, 128)
    vlin2 = _transpose_table(vT, tail2)
    out5d = _pair_gather(sT.astype(jnp.int32), vlin2)
    return out5d.transpose(2, 4, 0, 1, 3).reshape(4096, 200, _D)
